# baseline (speedup 1.0000x reference)
import functools

import jax
import jax.numpy as jnp
from jax import lax
from jax.experimental import pallas as pl
from jax.experimental.pallas import tpu as pltpu
from jax.experimental.pallas import tpu_sc as plsc

N = 10000
E = 320000
DIM = 128
G = 64

NC = 2
NS = 16

CHUNK = 128
TOTC = E // CHUNK
CPT = 160
CLAST = TOTC - (NS - 1) * CPT
CPAD = NS * CPT
NBUF = 4
NP = 10240
RPT = NP // NS

@functools.lru_cache(maxsize=None)
def _sc_mesh():
    return plsc.VectorSubcoreMesh(
        core_axis_name="c", subcore_axis_name="s",
        num_cores=NC, num_subcores=NS)


def _deg_body(rc_hbm, deg1_hbm, deg2_hbm,
              rb0, rb1, rb2, rb3, ones_v, zb_v, wb_v,
              si0, si1, si2, si3, ss0, ss1, ss2, ss3, deg_sh):
    c = lax.axis_index("c")
    s = lax.axis_index("s")
    rbs = (rb0, rb1, rb2, rb3)
    sis = (si0, si1, si2, si3)
    sss = (ss0, ss1, ss2, ss3)
    for j in range(CHUNK // 16):
        ones_v[pl.ds(j * 16, 16)] = jnp.ones((16,), jnp.float32)

    def _zero(i, carry):
        zb_v[pl.ds(i * 16, 16)] = jnp.zeros((16,), jnp.float32)
        return carry
    lax.fori_loop(0, 2000 // 16, _zero, 0)

    @pl.when(s < 5)
    def _():
        pltpu.sync_copy(zb_v, deg_sh.at[pl.ds(s * 2000, 2000)])
    plsc.subcore_barrier()

    cnt = jnp.where(s < NS - 1, CPT, CLAST)
    start = s * CPT

    for b in range(NBUF):
        pltpu.make_async_copy(rc_hbm.at[c, start + b, 0], rbs[b],
                              sis[b]).start()

    def _blk(j, carry):
        base = start + j * NBUF
        for b in range(NBUF):
            pltpu.make_async_copy(rc_hbm.at[c, base + b, 0], rbs[b],
                                  sis[b]).wait()
            pltpu.make_async_copy(ones_v, deg_sh.at[rbs[b]],
                                  sss[b]).start(add=True)
        for b in range(NBUF):
            k = base + b
            pltpu.make_async_copy(ones_v, deg_sh.at[rbs[b]], sss[b]).wait()

            @pl.when(k + NBUF - start < cnt)
            def _():
                pltpu.make_async_copy(rc_hbm.at[c, k + NBUF, 0], rbs[b],
                                      sis[b]).start()
        return carry
    lax.fori_loop(0, cnt // NBUF, _blk, 0)

    plsc.subcore_barrier()

    @pl.when(s < 10)
    def _():
        pltpu.sync_copy(deg_sh.at[pl.ds(s * 1000, 1000)], wb_v)

        @pl.when(c == 0)
        def _():
            pltpu.sync_copy(wb_v, deg1_hbm.at[pl.ds(s * 1000, 1000)])

        @pl.when(c == 1)
        def _():
            pltpu.sync_copy(wb_v, deg2_hbm.at[pl.ds(s * 1000, 1000)])


def _deg_call(rc_all):
    fn = pl.kernel(
        _deg_body,
        out_type=[jax.ShapeDtypeStruct((N,), jnp.float32),
                  jax.ShapeDtypeStruct((N,), jnp.float32)],
        mesh=_sc_mesh(),
        scratch_types=(
            [pltpu.VMEM((CHUNK,), jnp.int32)] * NBUF
            + [
                pltpu.VMEM((CHUNK,), jnp.float32),
                pltpu.VMEM((2000,), jnp.float32),
                pltpu.VMEM((1000,), jnp.float32),
            ]
            + [pltpu.SemaphoreType.DMA] * (2 * NBUF)
            + [pltpu.VMEM_SHARED((N,), jnp.float32)]
        ),
    )
    return fn(rc_all)


def _spmm_body(y_hbm, rc_hbm, t1_hbm, t2_hbm,
               rc0, rc1, rc2, rc3, gb0, gb1, rv0, rv1, t_sh,
               si0, si1, si2, si3, sg0, sg1):
    c = lax.axis_index("c")
    s = lax.axis_index("s")
    rcs = (rc0, rc1, rc2, rc3)
    gbs = (gb0, gb1)
    rvs = (rv0, rv1)
    sis = (si0, si1, si2, si3)
    sgs = (sg0, sg1)

    def _zero(i, carry):
        r = i // 8
        j = i - r * 8
        rv0[r, pl.ds(j * 16, 16)] = jnp.zeros((16,), jnp.float32)
        return carry
    lax.fori_loop(0, 128 * 8, _zero, 0)

    def _zcopy(k, carry):
        pltpu.sync_copy(rv0, t_sh.at[pl.ds(s * RPT + k * 128, 128)])
        return carry
    lax.fori_loop(0, RPT // 128, _zcopy, 0)
    plsc.subcore_barrier()

    cnt = jnp.where(s < NS - 1, CPT, CLAST)
    start = s * CPT
    y = y_hbm.at[c]

    for q in range(4):
        pltpu.make_async_copy(rc_hbm.at[c, start + q], rcs[q], sis[q]).start()

    def _stage_sidx(src_rc, b):
        for i in range(CHUNK // 16):
            gbs[b][pl.ds(i * 16, 16)] = src_rc[1, pl.ds(i * 16, 16)]

    def _blk(j, carry):
        base = start + 4 * j
        for q in range(4):
            b = q % 2
            l = 4 * j + q
            pltpu.make_async_copy(rc_hbm.at[c, base + q], rcs[q],
                                  sis[q]).wait()

            @pl.when(l >= 2)
            def _():
                pltpu.make_async_copy(rvs[b], t_sh.at[gbs[b]], sgs[b]).wait()
            _stage_sidx(rcs[q], b)
            pltpu.sync_copy(y.at[rcs[q].at[0]], rvs[b])
            pltpu.make_async_copy(rvs[b], t_sh.at[gbs[b]],
                                  sgs[b]).start(add=True)

            @pl.when(l + 4 < cnt)
            def _():
                pltpu.make_async_copy(rc_hbm.at[c, base + q + 4], rcs[q],
                                      sis[q]).start()
        return carry
    lax.fori_loop(0, cnt // 4, _blk, 0)

    for b in range(2):
        pltpu.make_async_copy(rvs[b], t_sh.at[gbs[b]], sgs[b]).wait()

    plsc.subcore_barrier()

    @pl.when(c == 0)
    def _():
        pltpu.sync_copy(t_sh.at[pl.ds(s * RPT, RPT)],
                        t1_hbm.at[pl.ds(s * RPT, RPT)])

    @pl.when(c == 1)
    def _():
        pltpu.sync_copy(t_sh.at[pl.ds(s * RPT, RPT)],
                        t2_hbm.at[pl.ds(s * RPT, RPT)])


def _spmm_call(y_all, rc_all):
    fn = pl.kernel(
        _spmm_body,
        out_type=[jax.ShapeDtypeStruct((NP, DIM), jnp.float32),
                  jax.ShapeDtypeStruct((NP, DIM), jnp.float32)],
        mesh=_sc_mesh(),
        scratch_types=(
            [pltpu.VMEM((2, CHUNK), jnp.int32)] * 4
            + [pltpu.VMEM((CHUNK,), jnp.int32)] * 2
            + [pltpu.VMEM((CHUNK, DIM), jnp.float32)] * 2
            + [pltpu.VMEM_SHARED((NP, DIM), jnp.float32)]
            + [pltpu.SemaphoreType.DMA] * 6
        ),
    )
    return fn(y_all, rc_all)


BLK = 1000
NB = N // BLK


def _dense_body(x_ref, lw_ref, lb_ref, w0_ref, w1_ref, deg_ref,
                h0_ref, y_ref, dis_ref):
    x = x_ref[0]
    out = jnp.maximum(
        jnp.dot(x, lw_ref[0], preferred_element_type=jnp.float32) + lb_ref[0],
        0.0)
    h0_ref[0] = jnp.dot(out, w0_ref[0], preferred_element_type=jnp.float32)
    deg = deg_ref[0]
    dis = jnp.where(deg > 0, lax.rsqrt(deg), 0.0)
    y_ref[0] = dis * jnp.dot(out, w1_ref[0], preferred_element_type=jnp.float32)
    dis_ref[0] = dis


def _dense_call(x, lw, lb, w0, w1, deg):
    return pl.pallas_call(
        _dense_body,
        grid=(2, NB),
        in_specs=[
            pl.BlockSpec((1, BLK, DIM), lambda c, i: (c, i, 0)),
            pl.BlockSpec((1, DIM, DIM), lambda c, i: (c, 0, 0)),
            pl.BlockSpec((1, 1, DIM), lambda c, i: (c, 0, 0)),
            pl.BlockSpec((1, DIM, DIM), lambda c, i: (c, 0, 0)),
            pl.BlockSpec((1, DIM, DIM), lambda c, i: (c, 0, 0)),
            pl.BlockSpec((1, BLK, 1), lambda c, i: (c, i, 0)),
        ],
        out_specs=[
            pl.BlockSpec((1, BLK, DIM), lambda c, i: (c, i, 0)),
            pl.BlockSpec((1, BLK, DIM), lambda c, i: (c, i, 0)),
            pl.BlockSpec((1, BLK, 1), lambda c, i: (c, i, 0)),
        ],
        out_shape=[
            jax.ShapeDtypeStruct((2, N, DIM), jnp.float32),
            jax.ShapeDtypeStruct((2, N, DIM), jnp.float32),
            jax.ShapeDtypeStruct((2, N, 1), jnp.float32),
        ],
    )(x, lw, lb, w0, w1, deg)


def _final_body(h0_ref, t_ref, dis_ref, cb_ref, oh_ref, fc2w_ref, fc2b_ref,
                out_ref, acc_ref):
    c = pl.program_id(0)
    i = pl.program_id(1)
    res = jnp.maximum(h0_ref[0] - dis_ref[0] * t_ref[0] + cb_ref[0], 0.0)
    part = lax.dot_general(oh_ref[0], res, (((0,), (0,)), ((), ())),
                           preferred_element_type=jnp.float32)

    @pl.when(i == 0)
    def _():
        acc_ref[c] = part

    @pl.when(i > 0)
    def _():
        acc_ref[c] = acc_ref[c] + part

    @pl.when((c == 1) & (i == NB - 1))
    def _():
        cat = jnp.concatenate([acc_ref[0], acc_ref[1]], axis=1)
        out_ref[...] = (
            jnp.dot(cat, fc2w_ref[...], preferred_element_type=jnp.float32)
            + fc2b_ref[0, 0])


def _final_call(h0, t, dis, cb, oh, fc2w, fc2b):
    return pl.pallas_call(
        _final_body,
        grid=(2, NB),
        in_specs=[
            pl.BlockSpec((1, BLK, DIM), lambda c, i: (c, i, 0)),
            pl.BlockSpec((1, BLK, DIM), lambda c, i: (c, i, 0)),
            pl.BlockSpec((1, BLK, 1), lambda c, i: (c, i, 0)),
            pl.BlockSpec((1, 1, DIM), lambda c, i: (c, 0, 0)),
            pl.BlockSpec((1, BLK, G), lambda c, i: (c, i, 0)),
            pl.BlockSpec((2 * DIM, 1), lambda c, i: (0, 0)),
            pl.BlockSpec((1, 1), lambda c, i: (0, 0)),
        ],
        out_specs=pl.BlockSpec((G, 1), lambda c, i: (0, 0)),
        out_shape=jax.ShapeDtypeStruct((G, 1), jnp.float32),
        scratch_shapes=[pltpu.VMEM((2, G, DIM), jnp.float32)],
    )(h0, t, dis, cb, oh, fc2w, fc2b)


def kernel(x1, x2, edge_index1, edge_index2, x1_batch, x2_batch,
           lin1_w, lin1_b, cheb1_w0, cheb1_w1, cheb1_b,
           lin2_w, lin2_b, cheb2_w0, cheb2_w1, cheb2_b,
           fc2_w, fc2_b):
    rc_all = jnp.stack([
        edge_index1.reshape(2, TOTC, CHUNK).transpose(1, 0, 2),
        edge_index2.reshape(2, TOTC, CHUNK).transpose(1, 0, 2),
    ])

    deg1, deg2 = _deg_call(rc_all)

    x = jnp.stack([x1, x2])
    lw = jnp.stack([lin1_w, lin2_w])
    lb = jnp.stack([lin1_b, lin2_b]).reshape(2, 1, DIM)
    w0 = jnp.stack([cheb1_w0, cheb2_w0])
    w1 = jnp.stack([cheb1_w1, cheb2_w1])
    deg = jnp.stack([deg1, deg2]).reshape(2, N, 1)

    h0, y, dis = _dense_call(x, lw, lb, w0, w1, deg)

    t1, t2 = _spmm_call(y, rc_all)
    t = jnp.stack([t1[:N], t2[:N]])

    cb = jnp.stack([cheb1_b, cheb2_b]).reshape(2, 1, DIM)
    gids = jnp.arange(G, dtype=x1_batch.dtype)
    oh = jnp.stack([
        (x1_batch[:, None] == gids[None, :]).astype(jnp.float32),
        (x2_batch[:, None] == gids[None, :]).astype(jnp.float32),
    ])

    pred = _final_call(h0, t, dis, cb, oh, fc2_w, fc2_b.reshape(1, 1))
    return pred.reshape(-1)


# speedup vs baseline: 1.2392x; 1.2392x over previous
import functools

import jax
import jax.numpy as jnp
from jax import lax
from jax.experimental import pallas as pl
from jax.experimental.pallas import tpu as pltpu
from jax.experimental.pallas import tpu_sc as plsc

N = 10000
E = 320000
DIM = 128
G = 64

NC = 2
NS = 16

CHUNK = 128
TOTC = E // CHUNK
CPT = 160
CLAST = TOTC - (NS - 1) * CPT
CPAD = NS * CPT
NBUF = 4
NP = 10240
RPT = NP // NS

@functools.lru_cache(maxsize=None)
def _sc_mesh():
    return plsc.VectorSubcoreMesh(
        core_axis_name="c", subcore_axis_name="s",
        num_cores=NC, num_subcores=NS)


def _deg_body(rc_hbm, deg1_hbm, deg2_hbm,
              rb0, rb1, rb2, rb3, ones_v, zb_v, wb_v,
              si0, si1, si2, si3, ss0, ss1, ss2, ss3, deg_sh):
    c = lax.axis_index("c")
    s = lax.axis_index("s")
    rbs = (rb0, rb1, rb2, rb3)
    sis = (si0, si1, si2, si3)
    sss = (ss0, ss1, ss2, ss3)
    for j in range(CHUNK // 16):
        ones_v[pl.ds(j * 16, 16)] = jnp.ones((16,), jnp.float32)

    def _zero(i, carry):
        zb_v[pl.ds(i * 16, 16)] = jnp.zeros((16,), jnp.float32)
        return carry
    lax.fori_loop(0, 2000 // 16, _zero, 0)

    @pl.when(s < 5)
    def _():
        pltpu.sync_copy(zb_v, deg_sh.at[pl.ds(s * 2000, 2000)])
    plsc.subcore_barrier()

    cnt = jnp.where(s < NS - 1, CPT, CLAST)
    start = s * CPT

    for b in range(NBUF):
        pltpu.make_async_copy(rc_hbm.at[c, start + b, 0], rbs[b],
                              sis[b]).start()

    def _blk(j, carry):
        base = start + j * NBUF
        for b in range(NBUF):
            pltpu.make_async_copy(rc_hbm.at[c, base + b, 0], rbs[b],
                                  sis[b]).wait()
            pltpu.make_async_copy(ones_v, deg_sh.at[rbs[b]],
                                  sss[b]).start(add=True)
        for b in range(NBUF):
            k = base + b
            pltpu.make_async_copy(ones_v, deg_sh.at[rbs[b]], sss[b]).wait()

            @pl.when(k + NBUF - start < cnt)
            def _():
                pltpu.make_async_copy(rc_hbm.at[c, k + NBUF, 0], rbs[b],
                                      sis[b]).start()
        return carry
    lax.fori_loop(0, cnt // NBUF, _blk, 0)

    plsc.subcore_barrier()

    @pl.when(s < 10)
    def _():
        pltpu.sync_copy(deg_sh.at[pl.ds(s * 1000, 1000)], wb_v)

        @pl.when(c == 0)
        def _():
            pltpu.sync_copy(wb_v, deg1_hbm.at[pl.ds(s * 1000, 1000)])

        @pl.when(c == 1)
        def _():
            pltpu.sync_copy(wb_v, deg2_hbm.at[pl.ds(s * 1000, 1000)])


def _deg_call(rc_all):
    fn = pl.kernel(
        _deg_body,
        out_type=[jax.ShapeDtypeStruct((N,), jnp.float32),
                  jax.ShapeDtypeStruct((N,), jnp.float32)],
        mesh=_sc_mesh(),
        scratch_types=(
            [pltpu.VMEM((CHUNK,), jnp.int32)] * NBUF
            + [
                pltpu.VMEM((CHUNK,), jnp.float32),
                pltpu.VMEM((2000,), jnp.float32),
                pltpu.VMEM((1000,), jnp.float32),
            ]
            + [pltpu.SemaphoreType.DMA] * (2 * NBUF)
            + [pltpu.VMEM_SHARED((N,), jnp.float32)]
        ),
    )
    return fn(rc_all)


def _spmm_body(y_hbm, rc_hbm, t_hbm,
               rc0, rc1, rc2, rc3, gb0, gb1, rv0, rv1, t_sh,
               si0, si1, si2, si3, sg0, sg1):
    c = lax.axis_index("c")
    s = lax.axis_index("s")
    rcs = (rc0, rc1, rc2, rc3)
    gbs = (gb0, gb1)
    rvs = (rv0, rv1)
    sis = (si0, si1, si2, si3)
    sgs = (sg0, sg1)

    def _zero(i, carry):
        r = i // 8
        j = i - r * 8
        rv0[r, pl.ds(j * 16, 16)] = jnp.zeros((16,), jnp.float32)
        return carry
    lax.fori_loop(0, 128 * 8, _zero, 0)

    def _zcopy(k, carry):
        pltpu.sync_copy(rv0, t_sh.at[pl.ds(s * RPT + k * 128, 128)])
        return carry
    lax.fori_loop(0, RPT // 128, _zcopy, 0)
    plsc.subcore_barrier()

    cnt = jnp.where(s < NS - 1, CPT, CLAST)
    start = s * CPT
    y = y_hbm.at[c]

    for q in range(4):
        pltpu.make_async_copy(rc_hbm.at[c, start + q], rcs[q], sis[q]).start()
    def _stage_gidx(src_rc, b):
        for i in range(CHUNK // 16):
            gbs[b][pl.ds(i * 16, 16)] = src_rc[0, pl.ds(i * 16, 16)]

    for b in range(2):
        pltpu.make_async_copy(rc_hbm.at[c, start + b], rcs[b], sis[b]).wait()
        _stage_gidx(rcs[b], b)
        pltpu.make_async_copy(y.at[gbs[b]], rvs[b], sgs[b]).start()

    def _blk(j, carry):
        base = start + 4 * j
        for q in range(4):
            b = q % 2
            l = 4 * j + q
            pltpu.make_async_copy(y.at[gbs[b]], rvs[b], sgs[b]).wait()
            pltpu.sync_copy(rvs[b], t_sh.at[rcs[q].at[1]], add=True)

            @pl.when(l + 4 < cnt)
            def _():
                pltpu.make_async_copy(rc_hbm.at[c, base + q + 4], rcs[q],
                                      sis[q]).start()

            @pl.when(l + 2 < cnt)
            def _():
                pltpu.make_async_copy(rc_hbm.at[c, base + q + 2],
                                      rcs[(q + 2) % 4], sis[(q + 2) % 4]).wait()
                _stage_gidx(rcs[(q + 2) % 4], b)
                pltpu.make_async_copy(y.at[gbs[b]], rvs[b], sgs[b]).start()
        return carry
    lax.fori_loop(0, cnt // 4, _blk, 0)

    plsc.subcore_barrier()

    pltpu.sync_copy(t_sh.at[pl.ds(s * RPT, RPT)],
                    t_hbm.at[c, pl.ds(s * RPT, RPT)])


def _spmm_call(y_all, rc_all):
    fn = pl.kernel(
        _spmm_body,
        out_type=jax.ShapeDtypeStruct((2, NP, DIM), jnp.float32),
        mesh=_sc_mesh(),
        scratch_types=(
            [pltpu.VMEM((2, CHUNK), jnp.int32)] * 4
            + [pltpu.VMEM((CHUNK,), jnp.int32)] * 2
            + [pltpu.VMEM((CHUNK, DIM), jnp.float32)] * 2
            + [pltpu.VMEM_SHARED((NP, DIM), jnp.float32)]
            + [pltpu.SemaphoreType.DMA] * 6
        ),
    )
    return fn(y_all, rc_all)


BLK = 2000
NB = N // BLK


def _dense_body(x_ref, lw_ref, lb_ref, w0_ref, w1_ref, deg_ref,
                h0_ref, y_ref):
    x = x_ref[0]
    out = jnp.maximum(
        jnp.dot(x, lw_ref[0], preferred_element_type=jnp.float32) + lb_ref[0],
        0.0)
    h0_ref[0] = jnp.dot(out, w0_ref[0], preferred_element_type=jnp.float32)
    deg = deg_ref[0]
    dis = jnp.where(deg > 0, lax.rsqrt(deg), 0.0)
    y_ref[0] = dis * jnp.dot(out, w1_ref[0], preferred_element_type=jnp.float32)


def _dense_call(x, lw, lb, w0, w1, deg):
    return pl.pallas_call(
        _dense_body,
        grid=(2, NB),
        in_specs=[
            pl.BlockSpec((1, BLK, DIM), lambda c, i: (c, i, 0)),
            pl.BlockSpec((1, DIM, DIM), lambda c, i: (c, 0, 0)),
            pl.BlockSpec((1, 1, DIM), lambda c, i: (c, 0, 0)),
            pl.BlockSpec((1, DIM, DIM), lambda c, i: (c, 0, 0)),
            pl.BlockSpec((1, DIM, DIM), lambda c, i: (c, 0, 0)),
            pl.BlockSpec((1, BLK, 1), lambda c, i: (c, i, 0)),
        ],
        out_specs=[
            pl.BlockSpec((1, BLK, DIM), lambda c, i: (c, i, 0)),
            pl.BlockSpec((1, BLK, DIM), lambda c, i: (c, i, 0)),
        ],
        out_shape=[
            jax.ShapeDtypeStruct((2, N, DIM), jnp.float32),
            jax.ShapeDtypeStruct((2, N, DIM), jnp.float32),
        ],
    )(x, lw, lb, w0, w1, deg)


def _final_body(h0_ref, t_ref, deg_ref, cb_ref, oh_ref, fc2w_ref, fc2b_ref,
                out_ref, acc_ref):
    c = pl.program_id(0)
    i = pl.program_id(1)
    deg = deg_ref[0]
    dis = jnp.where(deg > 0, lax.rsqrt(deg), 0.0)
    res = jnp.maximum(h0_ref[0] - dis * t_ref[0] + cb_ref[0], 0.0)
    part = lax.dot_general(oh_ref[0], res, (((0,), (0,)), ((), ())),
                           preferred_element_type=jnp.float32)

    @pl.when(i == 0)
    def _():
        acc_ref[c] = part

    @pl.when(i > 0)
    def _():
        acc_ref[c] = acc_ref[c] + part

    @pl.when((c == 1) & (i == NB - 1))
    def _():
        cat = jnp.concatenate([acc_ref[0], acc_ref[1]], axis=1)
        out_ref[...] = (
            jnp.dot(cat, fc2w_ref[...], preferred_element_type=jnp.float32)
            + fc2b_ref[0, 0])


def _final_call(h0, t, deg, cb, oh, fc2w, fc2b):
    return pl.pallas_call(
        _final_body,
        grid=(2, NB),
        in_specs=[
            pl.BlockSpec((1, BLK, DIM), lambda c, i: (c, i, 0)),
            pl.BlockSpec((1, BLK, DIM), lambda c, i: (c, i, 0)),
            pl.BlockSpec((1, BLK, 1), lambda c, i: (c, i, 0)),
            pl.BlockSpec((1, 1, DIM), lambda c, i: (c, 0, 0)),
            pl.BlockSpec((1, BLK, G), lambda c, i: (c, i, 0)),
            pl.BlockSpec((2 * DIM, 1), lambda c, i: (0, 0)),
            pl.BlockSpec((1, 1), lambda c, i: (0, 0)),
        ],
        out_specs=pl.BlockSpec((G, 1), lambda c, i: (0, 0)),
        out_shape=jax.ShapeDtypeStruct((G, 1), jnp.float32),
        scratch_shapes=[pltpu.VMEM((2, G, DIM), jnp.float32)],
    )(h0, t, deg, cb, oh, fc2w, fc2b)


def kernel(x1, x2, edge_index1, edge_index2, x1_batch, x2_batch,
           lin1_w, lin1_b, cheb1_w0, cheb1_w1, cheb1_b,
           lin2_w, lin2_b, cheb2_w0, cheb2_w1, cheb2_b,
           fc2_w, fc2_b):
    rc_all = jnp.stack([
        edge_index1.reshape(2, TOTC, CHUNK).transpose(1, 0, 2),
        edge_index2.reshape(2, TOTC, CHUNK).transpose(1, 0, 2),
    ])

    deg1, deg2 = _deg_call(rc_all)

    x = jnp.stack([x1, x2])
    lw = jnp.stack([lin1_w, lin2_w])
    lb = jnp.stack([lin1_b, lin2_b]).reshape(2, 1, DIM)
    w0 = jnp.stack([cheb1_w0, cheb2_w0])
    w1 = jnp.stack([cheb1_w1, cheb2_w1])
    deg = jnp.stack([deg1, deg2]).reshape(2, N, 1)

    h0, y = _dense_call(x, lw, lb, w0, w1, deg)

    t = _spmm_call(y, rc_all)

    cb = jnp.stack([cheb1_b, cheb2_b]).reshape(2, 1, DIM)
    gids = jnp.arange(G, dtype=x1_batch.dtype)
    oh = jnp.stack([
        (x1_batch[:, None] == gids[None, :]).astype(jnp.float32),
        (x2_batch[:, None] == gids[None, :]).astype(jnp.float32),
    ])

    pred = _final_call(h0, t, deg, cb, oh, fc2_w, fc2_b.reshape(1, 1))
    return pred.reshape(-1)
